# baseline (speedup 1.0000x reference)
import functools

import jax
import jax.numpy as jnp
from jax import lax
from jax.experimental import pallas as pl
from jax.experimental.pallas import tpu as pltpu
from jax.experimental.pallas import tpu_sc as plsc

INPUT_DIM = 2048
DICT_SIZE = 32768
K = 64
N_TOKENS = 128

_ENC_BD = 2048
_TOPK_BR = 16

_NC = 2
_NS = 16
_NW = _NC * _NS
_RPW = N_TOKENS // _NW
_L = 16


def _enc_body(x_ref, w_ref, b_ref, out_ref):
    out_ref[...] = jax.lax.dot_general(
        x_ref[...], w_ref[...],
        (((1,), (1,)), ((), ())),
        preferred_element_type=jnp.float32,
    ) + b_ref[...][None, :]


def _topk_body(pa_ref, acts_ref):
    v = pa_ref[...]
    bits = jax.lax.bitcast_convert_type(v, jnp.uint32)
    u = jnp.where(v >= 0.0, bits | jnp.uint32(0x80000000), ~bits)
    thr = jnp.zeros((v.shape[0], 1), jnp.uint32)
    for b in range(31, -1, -1):
        cand = thr | jnp.uint32(1 << b)
        cnt = jnp.sum((u >= cand).astype(jnp.int32), axis=1, keepdims=True)
        thr = jnp.where(cnt >= K, cand, thr)
    mask = u >= thr
    acts_ref[...] = jnp.where(mask, jnp.maximum(v, 0.0), 0.0)


def _sc_decode_body(acts_hbm, wdec_hbm, bdec_hbm, recon_hbm,
                    row_v, widx, wval, rowbuf, acc, bdec_v, sem0, sem1):
    wid = lax.axis_index("s") * _NC + lax.axis_index("c")
    pltpu.sync_copy(bdec_hbm, bdec_v)
    iota = lax.iota(jnp.int32, _L)
    zi = jnp.zeros((_L,), jnp.int32)
    zf = jnp.zeros((_L,), jnp.float32)
    sems = (sem0, sem1)

    def per_row(r0, carry):
        r = wid * _RPW + r0
        pltpu.sync_copy(acts_hbm.at[r], row_v)
        for j in range(K // _L):
            widx[pl.ds(j * _L, _L)] = zi
            wval[pl.ds(j * _L, _L)] = zf

        def scan_body(i, wcnt):
            v = row_v[pl.ds(i * _L, _L)]
            m = v != 0.0
            pos = jnp.maximum(wcnt + plsc.cumsum(m.astype(jnp.int32)) - 1, 0)
            plsc.store_scatter(widx, [pos], i * _L + iota, mask=m)
            plsc.store_scatter(wval, [pos], v, mask=m)
            return wcnt + plsc.all_reduce_population_count(m)

        lax.fori_loop(0, DICT_SIZE // _L, scan_body, zi)

        ngrp = K // _L
        cp = pltpu.async_copy(wdec_hbm.at[widx[pl.ds(0, _L)]],
                              rowbuf.at[0], sems[0])
        for o in range(INPUT_DIM // _L):
            acc[pl.ds(o * _L, _L)] = bdec_v[pl.ds(o * _L, _L)]
        for g in range(ngrp):
            cp.wait()
            if g + 1 < ngrp:
                cp = pltpu.async_copy(
                    wdec_hbm.at[widx[pl.ds((g + 1) * _L, _L)]],
                    rowbuf.at[(g + 1) % 2], sems[(g + 1) % 2])
            vchunk = wval[pl.ds(g * _L, _L)]
            vals = [jnp.broadcast_to(
                        jnp.sum(jnp.where(iota == j, vchunk, 0.0), axis=0),
                        (_L,))
                    for j in range(_L)]

            def acc_body(o, _, g=g, vals=vals):
                a = acc[pl.ds(o * _L, _L)]
                for j in range(_L):
                    a = a + vals[j] * rowbuf[g % 2, j, pl.ds(o * _L, _L)]
                acc[pl.ds(o * _L, _L)] = a
                return 0

            lax.fori_loop(0, INPUT_DIM // _L, acc_body, 0)
        pltpu.sync_copy(acc, recon_hbm.at[r])
        return carry

    lax.fori_loop(0, _RPW, per_row, 0)


def kernel(x, W_enc, b_enc, W_dec, b_dec):
    x_cent = x - b_dec[None, :]

    pre_acts = pl.pallas_call(
        _enc_body,
        grid=(DICT_SIZE // _ENC_BD,),
        in_specs=[
            pl.BlockSpec((N_TOKENS, INPUT_DIM), lambda d: (0, 0)),
            pl.BlockSpec((_ENC_BD, INPUT_DIM), lambda d: (d, 0)),
            pl.BlockSpec((_ENC_BD,), lambda d: (d,)),
        ],
        out_specs=pl.BlockSpec((N_TOKENS, _ENC_BD), lambda d: (0, d)),
        out_shape=jax.ShapeDtypeStruct((N_TOKENS, DICT_SIZE), jnp.float32),
    )(x_cent, W_enc, b_enc)

    acts = pl.pallas_call(
        _topk_body,
        grid=(N_TOKENS // _TOPK_BR,),
        in_specs=[pl.BlockSpec((_TOPK_BR, DICT_SIZE), lambda r: (r, 0))],
        out_specs=pl.BlockSpec((_TOPK_BR, DICT_SIZE), lambda r: (r, 0)),
        out_shape=jax.ShapeDtypeStruct((N_TOKENS, DICT_SIZE), jnp.float32),
    )(pre_acts)

    mesh = plsc.VectorSubcoreMesh(core_axis_name="c", subcore_axis_name="s",
                                  num_cores=_NC, num_subcores=_NS)
    recon = pl.kernel(
        _sc_decode_body,
        out_type=jax.ShapeDtypeStruct((N_TOKENS, INPUT_DIM), jnp.float32),
        mesh=mesh,
        compiler_params=pltpu.CompilerParams(needs_layout_passes=False),
        scratch_types=[
            pltpu.VMEM((DICT_SIZE,), jnp.float32),
            pltpu.VMEM((K,), jnp.int32),
            pltpu.VMEM((K,), jnp.float32),
            pltpu.VMEM((2, _L, INPUT_DIM), jnp.float32),
            pltpu.VMEM((INPUT_DIM,), jnp.float32),
            pltpu.VMEM((INPUT_DIM,), jnp.float32),
            pltpu.SemaphoreType.DMA,
            pltpu.SemaphoreType.DMA,
        ],
    )(acts, W_dec, b_dec)

    return (recon, acts)


# speedup vs baseline: 1.1661x; 1.1661x over previous
import functools

import jax
import jax.numpy as jnp
from jax import lax
from jax.experimental import pallas as pl
from jax.experimental.pallas import tpu as pltpu
from jax.experimental.pallas import tpu_sc as plsc

INPUT_DIM = 2048
DICT_SIZE = 32768
K = 64
N_TOKENS = 128

_ENC_BD = 2048
_TOPK_BR = 16
_BLK = 128
_NBLK = DICT_SIZE // _BLK

_NC = 2
_NS = 16
_NW = _NC * _NS
_RPW = N_TOKENS // _NW
_L = 16


def _enc_body(x_ref, w_ref, b_ref, out_ref):
    out_ref[...] = jax.lax.dot_general(
        x_ref[...], w_ref[...],
        (((1,), (1,)), ((), ())),
        preferred_element_type=jnp.float32,
    ) + b_ref[...][None, :]


def _topk_body(pa_ref, acts_ref, bmp_ref, thr_ref):
    v = pa_ref[...]
    bits = jax.lax.bitcast_convert_type(v, jnp.uint32)
    u = jnp.where(v >= 0.0, bits | jnp.uint32(0x80000000), ~bits)
    thr = jnp.zeros((v.shape[0], 1), jnp.uint32)
    for b in range(31, -1, -1):
        cand = thr | jnp.uint32(1 << b)
        cnt = jnp.sum((u >= cand).astype(jnp.int32), axis=1, keepdims=True)
        thr = jnp.where(cnt >= K, cand, thr)
    mask = u >= thr
    acts_ref[...] = jnp.where(mask, jnp.maximum(v, 0.0), 0.0)
    mi = mask.astype(jnp.int32).reshape(v.shape[0], _NBLK, _BLK)
    bmp_ref[...] = jnp.max(mi, axis=2)
    thr_f = jnp.where(
        thr >= jnp.uint32(0x80000000),
        jax.lax.bitcast_convert_type(thr & jnp.uint32(0x7FFFFFFF), jnp.float32),
        jax.lax.bitcast_convert_type(~thr, jnp.float32),
    )
    thr_ref[...] = jnp.broadcast_to(thr_f, (v.shape[0], _L))


def _sc_body(pab_hbm, bmp_hbm, thr_hbm, wdec_hbm, bdec_hbm, recon_hbm,
             bmv, thrv, blkids, cand, widx, wval, rowbuf, acc, bdec_v,
             sem0, sem1, sem2, sem3):
    wid = lax.axis_index("s") * _NC + lax.axis_index("c")
    pltpu.sync_copy(bdec_hbm, bdec_v)
    iota = lax.iota(jnp.int32, _L)
    zi = jnp.zeros((_L,), jnp.int32)
    sems = (sem0, sem1, sem2, sem3)
    nchunk_bm = _NBLK // _L
    ngrp = K // _L

    def per_row(r0, carry):
        r = wid * _RPW + r0
        pltpu.sync_copy(bmp_hbm.at[r], bmv)
        pltpu.sync_copy(thr_hbm.at[r], thrv)
        thr_s = jnp.broadcast_to(jnp.max(thrv[...], axis=0), (_L,))
        for j in range(K // _L):
            blkids[pl.ds(j * _L, _L)] = zi

        cnt = zi
        for c in range(nchunk_bm):
            m = bmv[pl.ds(c * _L, _L)] != 0
            pos = jnp.clip(cnt + plsc.cumsum(m.astype(jnp.int32)) - 1, 0, K - 1)
            plsc.store_scatter(blkids, [pos], c * _L + iota, mask=m)
            cnt = cnt + plsc.all_reduce_population_count(m)
        nblk = jnp.max(cnt, axis=0)

        base = jnp.broadcast_to(r * _NBLK, (_L,))
        cps = []
        for gch in range(K // _L):
            idxv = base + blkids[pl.ds(gch * _L, _L)]
            cps.append(pltpu.async_copy(
                pab_hbm.at[idxv], cand.at[pl.ds(gch * _L, _L)], sems[gch]))
        for cp in cps:
            cp.wait()

        zf = jnp.zeros((_L,), jnp.float32)
        for j in range(K // _L):
            widx[pl.ds(j * _L, _L)] = zi
            wval[pl.ds(j * _L, _L)] = zf

        def sel_block(b, wcnt):
            bch = b // _L
            bb = bch * _L
            blk_chunk = blkids[pl.ds(bb, _L)]
            blk_s = jnp.max(jnp.where(iota == b - bb, blk_chunk, 0), axis=0)
            colbase = jnp.broadcast_to(blk_s * _BLK, (_L,))
            for o in range(_BLK // _L):
                v = cand[b, pl.ds(o * _L, _L)]
                m = v >= thr_s
                pos = jnp.clip(
                    wcnt + plsc.cumsum(m.astype(jnp.int32)) - 1, 0, K - 1)
                plsc.store_scatter(widx, [pos], colbase + o * _L + iota, mask=m)
                plsc.store_scatter(wval, [pos], jnp.maximum(v, 0.0), mask=m)
                wcnt = wcnt + plsc.all_reduce_population_count(m)
            return wcnt

        lax.fori_loop(0, nblk, sel_block, zi)

        cp = pltpu.async_copy(wdec_hbm.at[widx[pl.ds(0, _L)]],
                              rowbuf.at[0], sems[0])
        for o in range(INPUT_DIM // _L):
            acc[pl.ds(o * _L, _L)] = bdec_v[pl.ds(o * _L, _L)]
        for g in range(ngrp):
            cp.wait()
            if g + 1 < ngrp:
                cp = pltpu.async_copy(
                    wdec_hbm.at[widx[pl.ds((g + 1) * _L, _L)]],
                    rowbuf.at[(g + 1) % 2], sems[(g + 1) % 2])
            vchunk = wval[pl.ds(g * _L, _L)]
            vals = [jnp.broadcast_to(
                        jnp.sum(jnp.where(iota == j, vchunk, 0.0), axis=0),
                        (_L,))
                    for j in range(_L)]

            def acc_body(o, _, g=g, vals=vals):
                a = acc[pl.ds(o * _L, _L)]
                for j in range(_L):
                    a = a + vals[j] * rowbuf[g % 2, j, pl.ds(o * _L, _L)]
                acc[pl.ds(o * _L, _L)] = a
                return 0

            lax.fori_loop(0, INPUT_DIM // _L, acc_body, 0)
        pltpu.sync_copy(acc, recon_hbm.at[r])
        return carry

    lax.fori_loop(0, _RPW, per_row, 0)


def kernel(x, W_enc, b_enc, W_dec, b_dec):
    x_cent = x - b_dec[None, :]

    pre_acts = pl.pallas_call(
        _enc_body,
        grid=(DICT_SIZE // _ENC_BD,),
        in_specs=[
            pl.BlockSpec((N_TOKENS, INPUT_DIM), lambda d: (0, 0)),
            pl.BlockSpec((_ENC_BD, INPUT_DIM), lambda d: (d, 0)),
            pl.BlockSpec((_ENC_BD,), lambda d: (d,)),
        ],
        out_specs=pl.BlockSpec((N_TOKENS, _ENC_BD), lambda d: (0, d)),
        out_shape=jax.ShapeDtypeStruct((N_TOKENS, DICT_SIZE), jnp.float32),
    )(x_cent, W_enc, b_enc)

    acts, bmp, thr = pl.pallas_call(
        _topk_body,
        grid=(N_TOKENS // _TOPK_BR,),
        in_specs=[pl.BlockSpec((_TOPK_BR, DICT_SIZE), lambda r: (r, 0))],
        out_specs=[
            pl.BlockSpec((_TOPK_BR, DICT_SIZE), lambda r: (r, 0)),
            pl.BlockSpec((_TOPK_BR, _NBLK), lambda r: (r, 0)),
            pl.BlockSpec((_TOPK_BR, _L), lambda r: (r, 0)),
        ],
        out_shape=[
            jax.ShapeDtypeStruct((N_TOKENS, DICT_SIZE), jnp.float32),
            jax.ShapeDtypeStruct((N_TOKENS, _NBLK), jnp.int32),
            jax.ShapeDtypeStruct((N_TOKENS, _L), jnp.float32),
        ],
    )(pre_acts)

    pa_blocks = pre_acts.reshape(N_TOKENS * _NBLK, _BLK)

    mesh = plsc.VectorSubcoreMesh(core_axis_name="c", subcore_axis_name="s",
                                  num_cores=_NC, num_subcores=_NS)
    recon = pl.kernel(
        _sc_body,
        out_type=jax.ShapeDtypeStruct((N_TOKENS, INPUT_DIM), jnp.float32),
        mesh=mesh,
        compiler_params=pltpu.CompilerParams(needs_layout_passes=False),
        scratch_types=[
            pltpu.VMEM((_NBLK,), jnp.int32),
            pltpu.VMEM((_L,), jnp.float32),
            pltpu.VMEM((K,), jnp.int32),
            pltpu.VMEM((K, _BLK), jnp.float32),
            pltpu.VMEM((K,), jnp.int32),
            pltpu.VMEM((K,), jnp.float32),
            pltpu.VMEM((2, _L, INPUT_DIM), jnp.float32),
            pltpu.VMEM((INPUT_DIM,), jnp.float32),
            pltpu.VMEM((INPUT_DIM,), jnp.float32),
            pltpu.SemaphoreType.DMA,
            pltpu.SemaphoreType.DMA,
            pltpu.SemaphoreType.DMA,
            pltpu.SemaphoreType.DMA,
        ],
    )(pa_blocks, bmp, thr, W_dec, b_dec)

    return (recon, acts)
